# Initial kernel scaffold; baseline (speedup 1.0000x reference)
import jax
import jax.numpy as jnp
from jax.experimental import pallas as pl


def kernel(token, weighted_factor, weighted_notes, src_table, wf_table, fp_W, fp_b, cp_W, cp_b):
    raise NotImplementedError("write your pallas kernel here")


# speedup vs baseline: 5.0836x; 5.0836x over previous
import functools

import jax
import jax.numpy as jnp
from jax import lax
from jax.experimental import pallas as pl
from jax.experimental.pallas import tpu as pltpu
from jax.experimental.pallas import tpu_sc as plsc

D = 128
VOCAB_BLK = 2000

N_TOK = 4096 * 200
NW = 32
PER_W = N_TOK // NW
C = 256
KI = C // 128
NCH = PER_W // C


def _transform_body(src_ref, wf_ref, cp_top_ref, cp_bot_ref, fpw_ref,
                    fpb_ref, cpb_ref, srcT_ref, wfT_ref):
    cp_top = cp_top_ref[...]
    cp_bot = cp_bot_ref[...]
    m = jnp.dot(fpw_ref[...], cp_bot, preferred_element_type=jnp.float32)
    bias = (jnp.dot(fpb_ref[0:1, :], cp_bot,
                    preferred_element_type=jnp.float32) + cpb_ref[0:1, :])
    srcT_ref[...] = jnp.dot(src_ref[...], cp_top,
                            preferred_element_type=jnp.float32) + bias
    wfT_ref[...] = jnp.dot(wf_ref[...], m,
                           preferred_element_type=jnp.float32)


def _transform_tables(src_table, wf_table, cp_top, cp_bot, fpw, fpb, cpb):
    vocab = src_table.shape[0]
    grid = (vocab // VOCAB_BLK,)
    blk = lambda r, c: pl.BlockSpec((r, c), lambda i: (0, 0))
    return pl.pallas_call(
        _transform_body,
        grid=grid,
        in_specs=[
            pl.BlockSpec((VOCAB_BLK, D), lambda i: (i, 0)),
            pl.BlockSpec((VOCAB_BLK, D), lambda i: (i, 0)),
            blk(128, 128), blk(32, 128), blk(128, 32), blk(8, 32),
            blk(8, 128),
        ],
        out_specs=[
            pl.BlockSpec((VOCAB_BLK, D), lambda i: (i, 0)),
            pl.BlockSpec((VOCAB_BLK, D), lambda i: (i, 0)),
        ],
        out_shape=[
            jax.ShapeDtypeStruct((vocab, D), jnp.float32),
            jax.ShapeDtypeStruct((vocab, D), jnp.float32),
        ],
    )(src_table, wf_table, cp_top, cp_bot, fpw, fpb, cpb)


def _sc_body(tok_hbm, wfi_hbm, wn_hbm, srcT_hbm, wfT_hbm, out_hbm,
             idx1, idx2, wnv, bufA, bufB, semA, semB):
    cid = lax.axis_index("c")
    sid = lax.axis_index("s")
    wid = sid * 2 + cid
    base = wid * PER_W
    base_r = wid * (PER_W // 128)

    lane = lax.broadcasted_iota(jnp.int32, (16,), 0)
    is_last = lane == 15

    def chunk(i, carry):
        r0 = base_r + i * KI
        off = pl.multiple_of(base + i * C, 256)
        pltpu.sync_copy(tok_hbm.at[pl.ds(r0, KI)], idx1)
        pltpu.sync_copy(wfi_hbm.at[pl.ds(r0, KI)], idx2)
        pltpu.sync_copy(wn_hbm.at[pl.ds(off, C)], wnv)
        cps = []
        for k in range(KI):
            cps.append(pltpu.async_copy(
                srcT_hbm.at[idx1.at[k]],
                bufA.at[pl.ds(k * 128, 128)], semA))
            cps.append(pltpu.async_copy(
                wfT_hbm.at[idx2.at[k]],
                bufB.at[pl.ds(k * 128, 128)], semB))
        for cp in cps:
            cp.wait()

        def grp_body(g, c2):
            wn_grp = wnv[pl.ds(g * 16, 16)]
            for j in range(16):
                r = g * 16 + j
                for c in range(D // 16):
                    s = pl.ds(c * 16, 16)
                    v = bufA[r, s] + bufB[r, s]
                    if c == D // 16 - 1:
                        v = jnp.where(is_last, wn_grp[j], v)
                    bufA[r, s] = v
            return c2
        lax.fori_loop(0, C // 16, grp_body, 0)

        pltpu.sync_copy(bufA, out_hbm.at[pl.ds(off, C)])
        return carry

    lax.fori_loop(0, NCH, chunk, 0)


def _sc_gather(tok, wfi, wn, srcT, wfT):
    mesh = plsc.VectorSubcoreMesh(core_axis_name="c", subcore_axis_name="s")
    f = functools.partial(
        pl.kernel, _sc_body, mesh=mesh,
        out_type=jax.ShapeDtypeStruct((N_TOK, D), jnp.float32),
        scratch_types=[
            pltpu.VMEM((KI, 128), jnp.int32),
            pltpu.VMEM((KI, 128), jnp.int32),
            pltpu.VMEM((C,), jnp.float32),
            pltpu.VMEM((C, D), jnp.float32),
            pltpu.VMEM((C, D), jnp.float32),
            pltpu.SemaphoreType.DMA,
            pltpu.SemaphoreType.DMA,
        ],
    )()
    return f(tok, wfi, wn, srcT, wfT)


def kernel(token, weighted_factor, weighted_notes, src_table, wf_table,
           fp_W, fp_b, cp_W, cp_b):
    tok = token.astype(jnp.int32).reshape(N_TOK // 128, 128)
    wfi = weighted_factor.astype(jnp.int32).reshape(N_TOK // 128, 128)
    wn = weighted_notes.astype(jnp.float32).reshape(N_TOK)

    cp_top = jnp.zeros((128, 128), jnp.float32).at[:, :127].set(cp_W[:128])
    cp_bot = jnp.zeros((32, 128), jnp.float32).at[:25, :127].set(cp_W[128:])
    fpw = jnp.zeros((128, 32), jnp.float32).at[:, :25].set(fp_W)
    fpb = jnp.zeros((8, 32), jnp.float32).at[0, :25].set(fp_b)
    cpb = jnp.zeros((8, 128), jnp.float32).at[0, :127].set(cp_b)

    srcT, wfT = _transform_tables(src_table, wf_table, cp_top, cp_bot,
                                  fpw, fpb, cpb)
    out = _sc_gather(tok, wfi, wn, srcT, wfT)
    return out.reshape(4096, 200, D)


# speedup vs baseline: 6.6439x; 1.3069x over previous
import functools

import jax
import jax.numpy as jnp
from jax import lax
from jax.experimental import pallas as pl
from jax.experimental.pallas import tpu as pltpu
from jax.experimental.pallas import tpu_sc as plsc

D = 128
VOCAB_BLK = 2000

N_TOK = 4096 * 200
NW = 32
PER_W = N_TOK // NW
C = 128
NCH = PER_W // C


def _transform_body(src_ref, wf_ref, cp_top_ref, cp_bot_ref, fpw_ref,
                    fpb_ref, cpb_ref, srcT_ref, wfT_ref):
    cp_top = cp_top_ref[...]
    cp_bot = cp_bot_ref[...]
    m = jnp.dot(fpw_ref[...], cp_bot, preferred_element_type=jnp.float32)
    bias = (jnp.dot(fpb_ref[0:1, :], cp_bot,
                    preferred_element_type=jnp.float32) + cpb_ref[0:1, :])
    srcT_ref[...] = jnp.dot(src_ref[...], cp_top,
                            preferred_element_type=jnp.float32) + bias
    wfT_ref[...] = jnp.dot(wf_ref[...], m,
                           preferred_element_type=jnp.float32)


def _transform_tables(src_table, wf_table, cp_top, cp_bot, fpw, fpb, cpb):
    vocab = src_table.shape[0]
    grid = (vocab // VOCAB_BLK,)
    blk = lambda r, c: pl.BlockSpec((r, c), lambda i: (0, 0))
    return pl.pallas_call(
        _transform_body,
        grid=grid,
        in_specs=[
            pl.BlockSpec((VOCAB_BLK, D), lambda i: (i, 0)),
            pl.BlockSpec((VOCAB_BLK, D), lambda i: (i, 0)),
            blk(128, 128), blk(32, 128), blk(128, 32), blk(8, 32),
            blk(8, 128),
        ],
        out_specs=[
            pl.BlockSpec((VOCAB_BLK, D), lambda i: (i, 0)),
            pl.BlockSpec((VOCAB_BLK, D), lambda i: (i, 0)),
        ],
        out_shape=[
            jax.ShapeDtypeStruct((vocab, D), jnp.float32),
            jax.ShapeDtypeStruct((vocab, D), jnp.float32),
        ],
    )(src_table, wf_table, cp_top, cp_bot, fpw, fpb, cpb)


def _sc_body(tok_hbm, wfi_hbm, wn_hbm, srcT_hbm, wfT_hbm, out_hbm,
             idxA0, idxA1, idxB0, idxB1, wn0, wn1,
             bufA0, bufA1, bufB0, bufB1, bufO0, bufO1,
             semG0, semG1, semW0, semW1):
    cid = lax.axis_index("c")
    sid = lax.axis_index("s")
    wid = sid * 2 + cid
    base = wid * PER_W
    base_r = wid * (PER_W // 128)

    lane = lax.broadcasted_iota(jnp.int32, (16,), 0)
    is_last = lane == 15

    idxA = [idxA0, idxA1]
    idxB = [idxB0, idxB1]
    wnv = [wn0, wn1]
    bufA = [bufA0, bufA1]
    bufB = [bufB0, bufB1]
    bufO = [bufO0, bufO1]
    semG = [semG0, semG1]
    semW = [semW0, semW1]

    def fire(s, c):
        r0 = base_r + c
        pltpu.sync_copy(tok_hbm.at[pl.ds(r0, 1)], idxA[s])
        pltpu.sync_copy(wfi_hbm.at[pl.ds(r0, 1)], idxB[s])
        pltpu.sync_copy(wn_hbm.at[pl.ds(r0, 1)], wnv[s])
        pltpu.async_copy(srcT_hbm.at[idxA[s].at[0]], bufA[s], semG[s])
        pltpu.async_copy(wfT_hbm.at[idxB[s].at[0]], bufB[s], semG[s])

    def wait_gathers(s):
        pltpu.make_async_copy(srcT_hbm.at[idxA[s].at[0]], bufA[s],
                              semG[s]).wait()
        pltpu.make_async_copy(wfT_hbm.at[idxB[s].at[0]], bufB[s],
                              semG[s]).wait()

    def compute(s):
        def grp_body(g, c2):
            wn_grp = wnv[s][0, pl.ds(g * 16, 16)]
            for j in range(16):
                r = g * 16 + j
                for c in range(D // 16):
                    sl = pl.ds(c * 16, 16)
                    v = bufA[s][r, sl] + bufB[s][r, sl]
                    if c == D // 16 - 1:
                        v = jnp.where(is_last, wn_grp[j], v)
                    bufO[s][r, sl] = v
            return c2
        lax.fori_loop(0, C // 16, grp_body, 0)

    def out_slice(c):
        return out_hbm.at[pl.ds(base + c * C, C)]

    fire(0, 0)
    fire(1, 1)

    def body2(t, carry):
        for s in range(2):
            c = 2 * t + s
            wait_gathers(s)
            @pl.when(t > 0)
            def _():
                pltpu.make_async_copy(bufO[s], out_slice(c), semW[s]).wait()
            compute(s)
            pltpu.async_copy(bufO[s], out_slice(c), semW[s])

            @pl.when(c + 2 < NCH)
            def _():
                fire(s, c + 2)
        return carry

    lax.fori_loop(0, NCH // 2, body2, 0)

    for s in range(2):
        pltpu.make_async_copy(bufO[s], out_slice(NCH - 2 + s),
                              semW[s]).wait()


def _sc_gather(tok, wfi, wn, srcT, wfT):
    mesh = plsc.VectorSubcoreMesh(core_axis_name="c", subcore_axis_name="s")
    f = functools.partial(
        pl.kernel, _sc_body, mesh=mesh,
        out_type=jax.ShapeDtypeStruct((N_TOK, D), jnp.float32),
        scratch_types=(
            [pltpu.VMEM((1, 128), jnp.int32)] * 4
            + [pltpu.VMEM((1, 128), jnp.float32)] * 2
            + [pltpu.VMEM((C, D), jnp.float32)] * 6
            + [pltpu.SemaphoreType.DMA] * 4
        ),
    )()
    return f(tok, wfi, wn, srcT, wfT)


def kernel(token, weighted_factor, weighted_notes, src_table, wf_table,
           fp_W, fp_b, cp_W, cp_b):
    tok = token.astype(jnp.int32).reshape(N_TOK // 128, 128)
    wfi = weighted_factor.astype(jnp.int32).reshape(N_TOK // 128, 128)
    wn = weighted_notes.astype(jnp.float32).reshape(N_TOK // 128, 128)

    cp_top = jnp.zeros((128, 128), jnp.float32).at[:, :127].set(cp_W[:128])
    cp_bot = jnp.zeros((32, 128), jnp.float32).at[:25, :127].set(cp_W[128:])
    fpw = jnp.zeros((128, 32), jnp.float32).at[:, :25].set(fp_W)
    fpb = jnp.zeros((8, 32), jnp.float32).at[0, :25].set(fp_b)
    cpb = jnp.zeros((8, 128), jnp.float32).at[0, :127].set(cp_b)

    srcT, wfT = _transform_tables(src_table, wf_table, cp_top, cp_bot,
                                  fpw, fpb, cpb)
    out = _sc_gather(tok, wfi, wn, srcT, wfT)
    return out.reshape(4096, 200, D)


# speedup vs baseline: 12.0510x; 1.8138x over previous
import functools

import jax
import jax.numpy as jnp
from jax import lax
from jax.experimental import pallas as pl
from jax.experimental.pallas import tpu as pltpu
from jax.experimental.pallas import tpu_sc as plsc

D = 128
VOCAB_BLK = 2000

N_TOK = 4096 * 200
NW = 32
PER_W = N_TOK // NW
C = 128
NCH = PER_W // C


def _transform_body(src_ref, wf_ref, cp_top_ref, cp_bot_ref, fpw_ref,
                    fpb_ref, cpb_ref, srcT_ref, wfT_ref):
    cp_top = cp_top_ref[...]
    cp_bot = cp_bot_ref[...]
    m = jnp.dot(fpw_ref[...], cp_bot, preferred_element_type=jnp.float32)
    bias = (jnp.dot(fpb_ref[0:1, :], cp_bot,
                    preferred_element_type=jnp.float32) + cpb_ref[0:1, :])
    srcT_ref[...] = jnp.dot(src_ref[...], cp_top,
                            preferred_element_type=jnp.float32) + bias
    wfT_ref[...] = jnp.dot(wf_ref[...], m,
                           preferred_element_type=jnp.float32)


def _transform_tables(src_table, wf_table, cp_top, cp_bot, fpw, fpb, cpb):
    vocab = src_table.shape[0]
    grid = (vocab // VOCAB_BLK,)
    blk = lambda r, c: pl.BlockSpec((r, c), lambda i: (0, 0))
    return pl.pallas_call(
        _transform_body,
        grid=grid,
        in_specs=[
            pl.BlockSpec((VOCAB_BLK, D), lambda i: (i, 0)),
            pl.BlockSpec((VOCAB_BLK, D), lambda i: (i, 0)),
            blk(128, 128), blk(32, 128), blk(128, 32), blk(8, 32),
            blk(8, 128),
        ],
        out_specs=[
            pl.BlockSpec((VOCAB_BLK, D), lambda i: (i, 0)),
            pl.BlockSpec((VOCAB_BLK, D), lambda i: (i, 0)),
        ],
        out_shape=[
            jax.ShapeDtypeStruct((vocab, D), jnp.float32),
            jax.ShapeDtypeStruct((vocab, D), jnp.float32),
        ],
    )(src_table, wf_table, cp_top, cp_bot, fpw, fpb, cpb)


NSET = 4


def _sc_body(tok_hbm, wfi_hbm, wn_hbm, srcT_hbm, wfT_hbm, out_hbm,
             idxA0, idxA1, idxA2, idxA3,
             idxB0, idxB1, idxB2, idxB3,
             wn0, wn1, wn2, wn3,
             bufG0, bufG1, bufG2, bufG3,
             semI0, semI1, semI2, semI3,
             semA0, semA1, semA2, semA3,
             semB0, semB1, semB2, semB3,
             semW0, semW1, semW2, semW3):
    cid = lax.axis_index("c")
    sid = lax.axis_index("s")
    wid = sid * 2 + cid
    base = wid * PER_W
    base_r = wid * (PER_W // 128)

    lane = lax.broadcasted_iota(jnp.int32, (16,), 0)
    is_last = lane == 15

    idxA = [idxA0, idxA1, idxA2, idxA3]
    idxB = [idxB0, idxB1, idxB2, idxB3]
    wnv = [wn0, wn1, wn2, wn3]
    bufG = [bufG0, bufG1, bufG2, bufG3]
    semI = [semI0, semI1, semI2, semI3]
    semA = [semA0, semA1, semA2, semA3]
    semB = [semB0, semB1, semB2, semB3]
    semW = [semW0, semW1, semW2, semW3]

    def out_slice(c):
        return out_hbm.at[pl.ds(base + c * C, C)]

    def fire_idx(s, c):
        r0 = base_r + c
        pltpu.async_copy(tok_hbm.at[pl.ds(r0, 1)], idxA[s], semI[s])
        pltpu.async_copy(wfi_hbm.at[pl.ds(r0, 1)], idxB[s], semI[s])
        pltpu.async_copy(wn_hbm.at[pl.ds(r0, 1)], wnv[s], semI[s])

    def wait_idx(s, c):
        r0 = base_r + c
        pltpu.make_async_copy(tok_hbm.at[pl.ds(r0, 1)], idxA[s],
                              semI[s]).wait()
        pltpu.make_async_copy(wfi_hbm.at[pl.ds(r0, 1)], idxB[s],
                              semI[s]).wait()
        pltpu.make_async_copy(wn_hbm.at[pl.ds(r0, 1)], wnv[s],
                              semI[s]).wait()

    def fire_a(s, c, first):
        if not first:
            @pl.when(c >= NSET)
            def _():
                pltpu.make_async_copy(bufG[s], out_slice(c), semW[s]).wait()
        wait_idx(s, c)
        pltpu.async_copy(srcT_hbm.at[idxA[s].at[0]], bufG[s], semA[s])

    def fire_b(s):
        pltpu.make_async_copy(srcT_hbm.at[idxA[s].at[0]], bufG[s],
                              semA[s]).wait()
        pltpu.async_copy(wfT_hbm.at[idxB[s].at[0]], bufG[s], semB[s],
                         add=True)

    def finish(s, c):
        pltpu.make_async_copy(wfT_hbm.at[idxB[s].at[0]], bufG[s],
                              semB[s]).wait()

        def grp_body(g, c2):
            wn_grp = wnv[s][0, pl.ds(g * 16, 16)]
            for j in range(16):
                r = g * 16 + j
                sl = pl.ds(D - 16, 16)
                bufG[s][r, sl] = jnp.where(is_last, wn_grp[j],
                                           bufG[s][r, sl])
            return c2
        lax.fori_loop(0, C // 16, grp_body, 0)
        pltpu.async_copy(bufG[s], out_slice(c), semW[s])

    fire_idx(0, 0)
    fire_idx(1, 1)
    fire_idx(2, 2)
    fire_a(0, 0, True)
    fire_b(0)
    fire_a(1, 1, True)

    def body4(t, carry):
        for u in range(NSET):
            i = NSET * t + u

            @pl.when(i + 3 < NCH)
            def _():
                fire_idx((u + 3) % NSET, i + 3)

            @pl.when(i + 2 < NCH)
            def _():
                fire_a((u + 2) % NSET, i + 2, False)

            @pl.when(i + 1 < NCH)
            def _():
                fire_b((u + 1) % NSET)

            finish(u, i)
        return carry

    lax.fori_loop(0, NCH // NSET, body4, 0)

    for u in range(NSET):
        pltpu.make_async_copy(bufG[u], out_slice(NCH - NSET + u),
                              semW[u]).wait()


def _sc_gather(tok, wfi, wn, srcT, wfT):
    mesh = plsc.VectorSubcoreMesh(core_axis_name="c", subcore_axis_name="s")
    f = functools.partial(
        pl.kernel, _sc_body, mesh=mesh,
        out_type=jax.ShapeDtypeStruct((N_TOK, D), jnp.float32),
        scratch_types=(
            [pltpu.VMEM((1, 128), jnp.int32)] * 8
            + [pltpu.VMEM((1, 128), jnp.float32)] * 4
            + [pltpu.VMEM((C, D), jnp.float32)] * 4
            + [pltpu.SemaphoreType.DMA] * 16
        ),
    )()
    return f(tok, wfi, wn, srcT, wfT)


def kernel(token, weighted_factor, weighted_notes, src_table, wf_table,
           fp_W, fp_b, cp_W, cp_b):
    tok = token.astype(jnp.int32).reshape(N_TOK // 128, 128)
    wfi = weighted_factor.astype(jnp.int32).reshape(N_TOK // 128, 128)
    wn = weighted_notes.astype(jnp.float32).reshape(N_TOK // 128, 128)

    cp_top = jnp.zeros((128, 128), jnp.float32).at[:, :127].set(cp_W[:128])
    cp_bot = jnp.zeros((32, 128), jnp.float32).at[:25, :127].set(cp_W[128:])
    fpw = jnp.zeros((128, 32), jnp.float32).at[:, :25].set(fp_W)
    fpb = jnp.zeros((8, 32), jnp.float32).at[0, :25].set(fp_b)
    cpb = jnp.zeros((8, 128), jnp.float32).at[0, :127].set(cp_b)

    srcT, wfT = _transform_tables(src_table, wf_table, cp_top, cp_bot,
                                  fpw, fpb, cpb)
    out = _sc_gather(tok, wfi, wn, srcT, wfT)
    return out.reshape(4096, 200, D)


# speedup vs baseline: 12.0885x; 1.0031x over previous
import functools

import jax
import jax.numpy as jnp
from jax import lax
from jax.experimental import pallas as pl
from jax.experimental.pallas import tpu as pltpu
from jax.experimental.pallas import tpu_sc as plsc

D = 128
VOCAB_BLK = 2000

N_TOK = 4096 * 200
NW = 32
PER_W = N_TOK // NW
C = 128
NCH = PER_W // C


def _transform_body(src_ref, wf_ref, cp_top_ref, cp_bot_ref, fpw_ref,
                    fpb_ref, cpb_ref, srcT_ref, wfT_ref):
    cp_top = cp_top_ref[...]
    cp_bot = cp_bot_ref[...]
    m = jnp.dot(fpw_ref[...], cp_bot, preferred_element_type=jnp.float32)
    bias = (jnp.dot(fpb_ref[0:1, :], cp_bot,
                    preferred_element_type=jnp.float32) + cpb_ref[0:1, :])
    srcT_ref[...] = jnp.dot(src_ref[...], cp_top,
                            preferred_element_type=jnp.float32) + bias
    wfT_ref[...] = jnp.dot(wf_ref[...], m,
                           preferred_element_type=jnp.float32)


def _transform_tables(src_table, wf_table, cp_top, cp_bot, fpw, fpb, cpb):
    vocab = src_table.shape[0]
    grid = (vocab // VOCAB_BLK,)
    blk = lambda r, c: pl.BlockSpec((r, c), lambda i: (0, 0))
    return pl.pallas_call(
        _transform_body,
        grid=grid,
        in_specs=[
            pl.BlockSpec((VOCAB_BLK, D), lambda i: (i, 0)),
            pl.BlockSpec((VOCAB_BLK, D), lambda i: (i, 0)),
            blk(128, 128), blk(32, 128), blk(128, 32), blk(8, 32),
            blk(8, 128),
        ],
        out_specs=[
            pl.BlockSpec((VOCAB_BLK, D), lambda i: (i, 0)),
            pl.BlockSpec((VOCAB_BLK, D), lambda i: (i, 0)),
        ],
        out_shape=[
            jax.ShapeDtypeStruct((vocab, D), jnp.float32),
            jax.ShapeDtypeStruct((vocab, D), jnp.float32),
        ],
    )(src_table, wf_table, cp_top, cp_bot, fpw, fpb, cpb)


NSET = 5


def _sc_body(tok_hbm, wfi_hbm, wn_hbm, srcT_hbm, wfT_hbm, out_hbm,
             idxA0, idxA1, idxA2, idxA3, idxA4,
             idxB0, idxB1, idxB2, idxB3, idxB4,
             wn0, wn1, wn2, wn3, wn4,
             bufG0, bufG1, bufG2, bufG3, bufG4,
             semI0, semI1, semI2, semI3, semI4,
             semA0, semA1, semA2, semA3, semA4,
             semB0, semB1, semB2, semB3, semB4,
             semW0, semW1, semW2, semW3, semW4):
    cid = lax.axis_index("c")
    sid = lax.axis_index("s")
    wid = sid * 2 + cid
    base = wid * PER_W
    base_r = wid * (PER_W // 128)

    lane = lax.broadcasted_iota(jnp.int32, (16,), 0)
    is_last = lane == 15

    idxA = [idxA0, idxA1, idxA2, idxA3, idxA4]
    idxB = [idxB0, idxB1, idxB2, idxB3, idxB4]
    wnv = [wn0, wn1, wn2, wn3, wn4]
    bufG = [bufG0, bufG1, bufG2, bufG3, bufG4]
    semI = [semI0, semI1, semI2, semI3, semI4]
    semA = [semA0, semA1, semA2, semA3, semA4]
    semB = [semB0, semB1, semB2, semB3, semB4]
    semW = [semW0, semW1, semW2, semW3, semW4]

    def out_slice(c):
        return out_hbm.at[pl.ds(base + c * C, C)]

    def fire_idx(s, c):
        r0 = base_r + c
        pltpu.async_copy(tok_hbm.at[pl.ds(r0, 1)], idxA[s], semI[s])
        pltpu.async_copy(wfi_hbm.at[pl.ds(r0, 1)], idxB[s], semI[s])
        pltpu.async_copy(wn_hbm.at[pl.ds(r0, 1)], wnv[s], semI[s])

    def wait_idx(s, c):
        r0 = base_r + c
        pltpu.make_async_copy(tok_hbm.at[pl.ds(r0, 1)], idxA[s],
                              semI[s]).wait()
        pltpu.make_async_copy(wfi_hbm.at[pl.ds(r0, 1)], idxB[s],
                              semI[s]).wait()
        pltpu.make_async_copy(wn_hbm.at[pl.ds(r0, 1)], wnv[s],
                              semI[s]).wait()

    def fire_a(s, c, first):
        if not first:
            @pl.when(c >= NSET)
            def _():
                pltpu.make_async_copy(bufG[s], out_slice(c), semW[s]).wait()
        wait_idx(s, c)
        pltpu.async_copy(srcT_hbm.at[idxA[s].at[0]], bufG[s], semA[s])

    def fire_b(s):
        pltpu.make_async_copy(srcT_hbm.at[idxA[s].at[0]], bufG[s],
                              semA[s]).wait()
        pltpu.async_copy(wfT_hbm.at[idxB[s].at[0]], bufG[s], semB[s],
                         add=True)

    def finish(s, c):
        pltpu.make_async_copy(wfT_hbm.at[idxB[s].at[0]], bufG[s],
                              semB[s]).wait()

        def grp_body(g, c2):
            wn_grp = wnv[s][0, pl.ds(g * 16, 16)]
            for j in range(16):
                r = g * 16 + j
                sl = pl.ds(D - 16, 16)
                bufG[s][r, sl] = jnp.where(is_last, wn_grp[j],
                                           bufG[s][r, sl])
            return c2
        lax.fori_loop(0, C // 16, grp_body, 0)
        pltpu.async_copy(bufG[s], out_slice(c), semW[s])

    fire_idx(0, 0)
    fire_idx(1, 1)
    fire_idx(2, 2)
    fire_idx(3, 3)
    fire_a(0, 0, True)
    fire_b(0)
    fire_a(1, 1, True)
    fire_a(2, 2, True)

    def body4(t, carry):
        for u in range(NSET):
            i = NSET * t + u

            @pl.when(i + 4 < NCH)
            def _():
                fire_idx((u + 4) % NSET, i + 4)

            @pl.when(i + 3 < NCH)
            def _():
                fire_a((u + 3) % NSET, i + 3, False)

            @pl.when(i + 1 < NCH)
            def _():
                fire_b((u + 1) % NSET)

            finish(u, i)
        return carry

    lax.fori_loop(0, NCH // NSET, body4, 0)

    for u in range(NSET):
        pltpu.make_async_copy(bufG[u], out_slice(NCH - NSET + u),
                              semW[u]).wait()


def _sc_gather(tok, wfi, wn, srcT, wfT):
    mesh = plsc.VectorSubcoreMesh(core_axis_name="c", subcore_axis_name="s")
    f = functools.partial(
        pl.kernel, _sc_body, mesh=mesh,
        out_type=jax.ShapeDtypeStruct((N_TOK, D), jnp.float32),
        scratch_types=(
            [pltpu.VMEM((1, 128), jnp.int32)] * 10
            + [pltpu.VMEM((1, 128), jnp.float32)] * 5
            + [pltpu.VMEM((C, D), jnp.float32)] * 5
            + [pltpu.SemaphoreType.DMA] * 20
        ),
    )()
    return f(tok, wfi, wn, srcT, wfT)


def kernel(token, weighted_factor, weighted_notes, src_table, wf_table,
           fp_W, fp_b, cp_W, cp_b):
    tok = token.astype(jnp.int32).reshape(N_TOK // 128, 128)
    wfi = weighted_factor.astype(jnp.int32).reshape(N_TOK // 128, 128)
    wn = weighted_notes.astype(jnp.float32).reshape(N_TOK // 128, 128)

    cp_top = jnp.zeros((128, 128), jnp.float32).at[:, :127].set(cp_W[:128])
    cp_bot = jnp.zeros((32, 128), jnp.float32).at[:25, :127].set(cp_W[128:])
    fpw = jnp.zeros((128, 32), jnp.float32).at[:, :25].set(fp_W)
    fpb = jnp.zeros((8, 32), jnp.float32).at[0, :25].set(fp_b)
    cpb = jnp.zeros((8, 128), jnp.float32).at[0, :127].set(cp_b)

    srcT, wfT = _transform_tables(src_table, wf_table, cp_top, cp_bot,
                                  fpw, fpb, cpb)
    out = _sc_gather(tok, wfi, wn, srcT, wfT)
    return out.reshape(4096, 200, D)
